# baseline (speedup 1.0000x reference)
import functools

import jax
import jax.numpy as jnp
from jax import lax
from jax.experimental import pallas as pl
from jax.experimental.pallas import tpu as pltpu
from jax.experimental.pallas import tpu_sc as plsc

N = 10000
D = 128
NW = 32
B = 128

KDEG = 80
NPD = 10112
SDEG = NPD // 16
EP = NW * KDEG * B

THIRD = 3456
YPAD = 3 * THIRD
TSTRIPE = THIRD // 16
DSTH = 5000
ACCR = 5248
ASTRIPE = ACCR // 16
GARB = 5040
C = 28

_MESH = plsc.VectorSubcoreMesh(core_axis_name="c", subcore_axis_name="s")


@functools.partial(
    pl.kernel,
    out_type=jax.ShapeDtypeStruct((2, NPD, D), jnp.float32),
    mesh=_MESH,
    scratch_types=[
        pltpu.VMEM((KDEG, B), jnp.int32),
        pltpu.VMEM((B, D), jnp.float32),
        pltpu.VMEM_SHARED((NPD, D), jnp.float32),
    ],
)
def _deg_kernel(dst_hbm, ones_hbm, zeros_hbm, out_hbm, dst_v, ones_v, acc):
    cid = lax.axis_index("c")
    sid = lax.axis_index("s")
    wid = cid * 16 + sid
    pltpu.sync_copy(dst_hbm.at[wid], dst_v)
    pltpu.sync_copy(ones_hbm, ones_v)
    pltpu.sync_copy(zeros_hbm, acc.at[pl.ds(sid * SDEG, SDEG)])
    plsc.subcore_barrier()

    @pl.loop(0, KDEG)
    def _(c):
        pltpu.sync_copy(ones_v, acc.at[dst_v.at[c]], add=True)

    plsc.subcore_barrier()
    pltpu.sync_copy(acc.at[pl.ds(sid * SDEG, SDEG)],
                    out_hbm.at[cid, pl.ds(sid * SDEG, SDEG)])


@functools.partial(
    pl.kernel,
    out_type=jax.ShapeDtypeStruct((2, ACCR, D), jnp.float32),
    mesh=_MESH,
    scratch_types=[
        pltpu.VMEM((C, B), jnp.int32),
        pltpu.VMEM((C, B), jnp.int32),
        pltpu.VMEM((B, D), jnp.float32),
        pltpu.VMEM((B, D), jnp.float32),
        pltpu.VMEM_SHARED((THIRD, D), jnp.float32),
        pltpu.VMEM_SHARED((ACCR, D), jnp.float32),
        pltpu.SemaphoreType.DMA,
        pltpu.SemaphoreType.DMA,
    ],
)
def _scatter_kernel(y_hbm, src_hbm, dst_hbm, zeros_hbm, out_hbm,
                    src_v, dst_v, rows0, rows1, ysh, acc, sem0, sem1):
    cid = lax.axis_index("c")
    sid = lax.axis_index("s")
    pltpu.sync_copy(zeros_hbm, acc.at[pl.ds(sid * ASTRIPE, ASTRIPE)])
    rows = (rows0, rows1)
    sems = (sem0, sem1)

    for p in range(3):
        pltpu.sync_copy(y_hbm.at[pl.ds(p * THIRD + sid * TSTRIPE, TSTRIPE)],
                        ysh.at[pl.ds(sid * TSTRIPE, TSTRIPE)])
        pltpu.sync_copy(src_hbm.at[p, cid, sid], src_v)
        pltpu.sync_copy(dst_hbm.at[p, cid, sid], dst_v)
        plsc.subcore_barrier()

        pltpu.async_copy(ysh.at[src_v.at[0]], rows0, sem0)
        pltpu.async_copy(ysh.at[src_v.at[1]], rows1, sem1)

        @pl.loop(0, C - 2, step=2)
        def _(g):
            for b in range(2):
                c = g + b
                pltpu.make_async_copy(ysh.at[src_v.at[c]], rows[b],
                                      sems[b]).wait()
                pltpu.sync_copy(rows[b], acc.at[dst_v.at[c]], add=True)
                pltpu.async_copy(ysh.at[src_v.at[c + 2]], rows[b], sems[b])

        for b in range(2):
            c = C - 2 + b
            pltpu.make_async_copy(ysh.at[src_v.at[c]], rows[b],
                                  sems[b]).wait()
            pltpu.sync_copy(rows[b], acc.at[dst_v.at[c]], add=True)

        plsc.subcore_barrier()

    pltpu.sync_copy(acc.at[pl.ds(sid * ASTRIPE, ASTRIPE)],
                    out_hbm.at[cid, pl.ds(sid * ASTRIPE, ASTRIPE)])


GW = 16


@functools.partial(
    pl.kernel,
    out_type=jax.ShapeDtypeStruct((2, NPD, D), jnp.float32),
    mesh=_MESH,
    scratch_types=[
        pltpu.VMEM((GW, B), jnp.int32),
        pltpu.VMEM((GW, B), jnp.int32),
        pltpu.VMEM((B, D), jnp.float32),
        pltpu.VMEM((B, D), jnp.float32),
        pltpu.VMEM_SHARED((NPD, D), jnp.float32),
        pltpu.SemaphoreType.DMA,
        pltpu.SemaphoreType.DMA,
    ],
)
def _scatter_wide(y_hbm, src_hbm, dst_hbm, zeros_hbm, out_hbm,
                  src_v, dst_v, rows0, rows1, acc, sem0, sem1):
    cid = lax.axis_index("c")
    sid = lax.axis_index("s")
    wid = cid * 16 + sid
    pltpu.sync_copy(zeros_hbm, acc.at[pl.ds(sid * SDEG, SDEG)])
    plsc.subcore_barrier()

    rows = (rows0, rows1)
    sems = (sem0, sem1)
    for grp in range(KDEG // GW):
        pltpu.sync_copy(src_hbm.at[wid, pl.ds(grp * GW, GW)], src_v)
        pltpu.sync_copy(dst_hbm.at[wid, pl.ds(grp * GW, GW)], dst_v)
        pltpu.async_copy(y_hbm.at[src_v.at[0]], rows0, sem0)
        pltpu.async_copy(y_hbm.at[src_v.at[1]], rows1, sem1)

        @pl.loop(0, GW - 2, step=2)
        def _(g):
            for b in range(2):
                c = g + b
                pltpu.make_async_copy(y_hbm.at[src_v.at[c]], rows[b],
                                      sems[b]).wait()
                pltpu.sync_copy(rows[b], acc.at[dst_v.at[c]], add=True)
                pltpu.async_copy(y_hbm.at[src_v.at[c + 2]], rows[b], sems[b])

        for b in range(2):
            c = GW - 2 + b
            pltpu.make_async_copy(y_hbm.at[src_v.at[c]], rows[b],
                                  sems[b]).wait()
            pltpu.sync_copy(rows[b], acc.at[dst_v.at[c]], add=True)

    plsc.subcore_barrier()
    pltpu.sync_copy(acc.at[pl.ds(sid * SDEG, SDEG)],
                    out_hbm.at[cid, pl.ds(sid * SDEG, SDEG)])


def _bn_relu_w(aggp_ref, y_ref, degp_ref, cb_ref, g_ref, bb_ref):
    dinv = _deg_inv(degp_ref)
    t = dinv * (aggp_ref[0, :N, :] + aggp_ref[1, :N, :] + y_ref[:N, :]) + cb_ref[...]
    mu = jnp.mean(t, axis=0, keepdims=True)
    dev = t - mu
    var = jnp.mean(dev * dev, axis=0, keepdims=True)
    h = jnp.maximum(dev * lax.rsqrt(var + 1e-5) * g_ref[...] + bb_ref[...], 0.0)
    return h, dinv


def _layer_body_w(aggp_ref, y_ref, degp_ref, cb_ref, g_ref, bb_ref, wn_ref, out_ref):
    h, dinv = _bn_relu_w(aggp_ref, y_ref, degp_ref, cb_ref, g_ref, bb_ref)
    yn = jnp.dot(dinv * h, wn_ref[...], preferred_element_type=jnp.float32)
    out_ref[...] = jnp.concatenate(
        [yn, jnp.zeros((YPAD - N, D), jnp.float32)], axis=0)


def _last_body_w(aggp_ref, y_ref, degp_ref, cb_ref, g_ref, bb_ref,
                 wout_ref, bout_ref, out_ref):
    h, _ = _bn_relu_w(aggp_ref, y_ref, degp_ref, cb_ref, g_ref, bb_ref)
    s = jnp.sum(h, axis=0, keepdims=True) * (1.0 / N)
    out_ref[...] = jnp.dot(s, wout_ref[...],
                           preferred_element_type=jnp.float32) + bout_ref[...]


def _deg_inv(degp_ref):
    deg = degp_ref[0, :N, 0:1] + degp_ref[1, :N, 0:1] + 1.0
    return lax.rsqrt(deg)


def _agg(aggp_ref):
    return jnp.concatenate(
        [aggp_ref[0, :DSTH, :], aggp_ref[1, :N - DSTH, :]], axis=0)


def _emb_body(x_ref, wemb_ref, bemb_ref, w0_ref, degp_ref, y0_ref):
    h = jnp.dot(x_ref[...], wemb_ref[...],
                preferred_element_type=jnp.float32) + bemb_ref[...]
    xw = jnp.dot(h, w0_ref[...], preferred_element_type=jnp.float32)
    y0 = xw * _deg_inv(degp_ref)
    y0_ref[...] = jnp.concatenate(
        [y0, jnp.zeros((YPAD - N, D), jnp.float32)], axis=0)


def _bn_relu(aggp_ref, y_ref, degp_ref, cb_ref, g_ref, bb_ref):
    dinv = _deg_inv(degp_ref)
    t = dinv * (_agg(aggp_ref) + y_ref[:N, :]) + cb_ref[...]
    mu = jnp.mean(t, axis=0, keepdims=True)
    dev = t - mu
    var = jnp.mean(dev * dev, axis=0, keepdims=True)
    h = jnp.maximum(dev * lax.rsqrt(var + 1e-5) * g_ref[...] + bb_ref[...], 0.0)
    return h, dinv


def _layer_body(aggp_ref, y_ref, degp_ref, cb_ref, g_ref, bb_ref, wn_ref, out_ref):
    h, dinv = _bn_relu(aggp_ref, y_ref, degp_ref, cb_ref, g_ref, bb_ref)
    yn = jnp.dot(dinv * h, wn_ref[...], preferred_element_type=jnp.float32)
    out_ref[...] = jnp.concatenate(
        [yn, jnp.zeros((YPAD - N, D), jnp.float32)], axis=0)


def _last_body(aggp_ref, y_ref, degp_ref, cb_ref, g_ref, bb_ref,
               wout_ref, bout_ref, out_ref):
    h, _ = _bn_relu(aggp_ref, y_ref, degp_ref, cb_ref, g_ref, bb_ref)
    s = jnp.sum(h, axis=0, keepdims=True) * (1.0 / N)
    out_ref[...] = jnp.dot(s, wout_ref[...],
                           preferred_element_type=jnp.float32) + bout_ref[...]


def kernel(x, edge_index, W_emb, b_emb, conv_w, conv_b, bn_g, bn_b, W_out, b_out):
    src = edge_index[0]
    dst = edge_index[1]
    E = src.shape[0]

    pad = EP - E
    dst_t = jnp.concatenate([dst, jnp.full((pad,), N, jnp.int32)]).reshape(
        NW, KDEG, B)
    onesD = jnp.ones((B, D), jnp.float32)
    zerosDeg = jnp.zeros((SDEG, D), jnp.float32)
    zerosAcc = jnp.zeros((ASTRIPE, D), jnp.float32)

    third = src // THIRD
    half = (dst >= DSTH).astype(jnp.int32)
    bucket = third * 2 + half
    src_local = src - third * THIRD
    dst_local = dst - half * DSTH
    cap = 16 * C * B
    rank = jnp.zeros((E,), jnp.int32)
    counts = []
    for bkt in range(6):
        m = bucket == bkt
        r = jnp.cumsum(m.astype(jnp.int32))
        rank = jnp.where(m, r - 1, rank)
        counts.append(r[-1])
    counts = jnp.stack(counts)
    slot = jnp.minimum(bucket * cap + rank, bucket * cap + cap - 1)
    src_r = jnp.zeros((6 * cap,), jnp.int32).at[slot].set(
        src_local).reshape(3, 2, 16, C, B)
    dst_r = jnp.full((6 * cap,), GARB, jnp.int32).at[slot].set(
        dst_local).reshape(3, 2, 16, C, B)

    src_t = jnp.concatenate([src, jnp.zeros((pad,), jnp.int32)]).reshape(
        NW, KDEG, B)

    degp = _deg_kernel(dst_t, onesD, zerosDeg)

    y0 = pl.pallas_call(
        _emb_body, out_shape=jax.ShapeDtypeStruct((YPAD, D), jnp.float32))(
            x, W_emb, b_emb.reshape(1, D), conv_w[0], degp)

    def _fast(y):
        out = None
        for l in range(4):
            aggp = _scatter_kernel(y, src_r, dst_r, zerosAcc)
            if l < 3:
                y = pl.pallas_call(
                    _layer_body,
                    out_shape=jax.ShapeDtypeStruct((YPAD, D), jnp.float32))(
                        aggp, y, degp, conv_b[l].reshape(1, D),
                        bn_g[l].reshape(1, D), bn_b[l].reshape(1, D),
                        conv_w[l + 1])
            else:
                out = pl.pallas_call(
                    _last_body,
                    out_shape=jax.ShapeDtypeStruct((1, 1), jnp.float32))(
                        aggp, y, degp, conv_b[l].reshape(1, D),
                        bn_g[l].reshape(1, D), bn_b[l].reshape(1, D),
                        W_out, b_out.reshape(1, 1))
        return out

    def _safe(y):
        out = None
        for l in range(4):
            aggp = _scatter_wide(y, src_t, dst_t, zerosDeg)
            if l < 3:
                y = pl.pallas_call(
                    _layer_body_w,
                    out_shape=jax.ShapeDtypeStruct((YPAD, D), jnp.float32))(
                        aggp, y, degp, conv_b[l].reshape(1, D),
                        bn_g[l].reshape(1, D), bn_b[l].reshape(1, D),
                        conv_w[l + 1])
            else:
                out = pl.pallas_call(
                    _last_body_w,
                    out_shape=jax.ShapeDtypeStruct((1, 1), jnp.float32))(
                        aggp, y, degp, conv_b[l].reshape(1, D),
                        bn_g[l].reshape(1, D), bn_b[l].reshape(1, D),
                        W_out, b_out.reshape(1, 1))
        return out

    fits = jnp.all(counts <= cap)
    return lax.cond(fits, _fast, _safe, y0)


# speedup vs baseline: 2.4827x; 2.4827x over previous
import functools

import jax
import jax.numpy as jnp
from jax import lax
from jax.experimental import pallas as pl
from jax.experimental.pallas import tpu as pltpu
from jax.experimental.pallas import tpu_sc as plsc

N = 10000
D = 128
NW = 32
B = 128

KDEG = 80
NPD = 10112
SDEG = NPD // 16
EP = NW * KDEG * B

THIRD = 3456
YPAD = 3 * THIRD
TSTRIPE = THIRD // 16
DSTH = 5000
ACCR = 5248
ASTRIPE = ACCR // 16
GARB = 5040
C = 28

_MESH = plsc.VectorSubcoreMesh(core_axis_name="c", subcore_axis_name="s")


@functools.partial(
    pl.kernel,
    out_type=jax.ShapeDtypeStruct((2, NPD, D), jnp.float32),
    mesh=_MESH,
    scratch_types=[
        pltpu.VMEM((KDEG, B), jnp.int32),
        pltpu.VMEM((B, D), jnp.float32),
        pltpu.VMEM_SHARED((NPD, D), jnp.float32),
    ],
)
def _deg_kernel(dst_hbm, ones_hbm, zeros_hbm, out_hbm, dst_v, ones_v, acc):
    cid = lax.axis_index("c")
    sid = lax.axis_index("s")
    wid = cid * 16 + sid
    pltpu.sync_copy(dst_hbm.at[wid], dst_v)
    pltpu.sync_copy(ones_hbm, ones_v)
    pltpu.sync_copy(zeros_hbm, acc.at[pl.ds(sid * SDEG, SDEG)])
    plsc.subcore_barrier()

    @pl.loop(0, KDEG)
    def _(c):
        pltpu.sync_copy(ones_v, acc.at[dst_v.at[c]], add=True)

    plsc.subcore_barrier()
    pltpu.sync_copy(acc.at[pl.ds(sid * SDEG, SDEG)],
                    out_hbm.at[cid, pl.ds(sid * SDEG, SDEG)])


@functools.partial(
    pl.kernel,
    out_type=jax.ShapeDtypeStruct((2, ACCR, D), jnp.float32),
    mesh=_MESH,
    scratch_types=[
        pltpu.VMEM((C, B), jnp.int32),
        pltpu.VMEM((C, B), jnp.int32),
        pltpu.VMEM((B, D), jnp.float32),
        pltpu.VMEM((B, D), jnp.float32),
        pltpu.VMEM_SHARED((THIRD, D), jnp.float32),
        pltpu.VMEM_SHARED((ACCR, D), jnp.float32),
        pltpu.SemaphoreType.DMA,
        pltpu.SemaphoreType.DMA,
    ],
)
def _scatter_kernel(y_hbm, src_hbm, dst_hbm, zeros_hbm, out_hbm,
                    src_v, dst_v, rows0, rows1, ysh, acc, sem0, sem1):
    cid = lax.axis_index("c")
    sid = lax.axis_index("s")
    pltpu.sync_copy(zeros_hbm, acc.at[pl.ds(sid * ASTRIPE, ASTRIPE)])
    rows = (rows0, rows1)
    sems = (sem0, sem1)

    for p in range(3):
        pltpu.sync_copy(y_hbm.at[pl.ds(p * THIRD + sid * TSTRIPE, TSTRIPE)],
                        ysh.at[pl.ds(sid * TSTRIPE, TSTRIPE)])
        pltpu.sync_copy(src_hbm.at[p, cid, sid], src_v)
        pltpu.sync_copy(dst_hbm.at[p, cid, sid], dst_v)
        plsc.subcore_barrier()

        pltpu.async_copy(ysh.at[src_v.at[0]], rows0, sem0)
        pltpu.async_copy(ysh.at[src_v.at[1]], rows1, sem1)

        @pl.loop(0, C - 2, step=2)
        def _(g):
            for b in range(2):
                c = g + b
                pltpu.make_async_copy(ysh.at[src_v.at[c]], rows[b],
                                      sems[b]).wait()
                pltpu.sync_copy(rows[b], acc.at[dst_v.at[c]], add=True)
                pltpu.async_copy(ysh.at[src_v.at[c + 2]], rows[b], sems[b])

        for b in range(2):
            c = C - 2 + b
            pltpu.make_async_copy(ysh.at[src_v.at[c]], rows[b],
                                  sems[b]).wait()
            pltpu.sync_copy(rows[b], acc.at[dst_v.at[c]], add=True)

        plsc.subcore_barrier()

    pltpu.sync_copy(acc.at[pl.ds(sid * ASTRIPE, ASTRIPE)],
                    out_hbm.at[cid, pl.ds(sid * ASTRIPE, ASTRIPE)])


GW = 16


@functools.partial(
    pl.kernel,
    out_type=jax.ShapeDtypeStruct((2, NPD, D), jnp.float32),
    mesh=_MESH,
    scratch_types=[
        pltpu.VMEM((GW, B), jnp.int32),
        pltpu.VMEM((GW, B), jnp.int32),
        pltpu.VMEM((B, D), jnp.float32),
        pltpu.VMEM((B, D), jnp.float32),
        pltpu.VMEM_SHARED((NPD, D), jnp.float32),
        pltpu.SemaphoreType.DMA,
        pltpu.SemaphoreType.DMA,
    ],
)
def _scatter_wide(y_hbm, src_hbm, dst_hbm, zeros_hbm, out_hbm,
                  src_v, dst_v, rows0, rows1, acc, sem0, sem1):
    cid = lax.axis_index("c")
    sid = lax.axis_index("s")
    wid = cid * 16 + sid
    pltpu.sync_copy(zeros_hbm, acc.at[pl.ds(sid * SDEG, SDEG)])
    plsc.subcore_barrier()

    rows = (rows0, rows1)
    sems = (sem0, sem1)
    for grp in range(KDEG // GW):
        pltpu.sync_copy(src_hbm.at[wid, pl.ds(grp * GW, GW)], src_v)
        pltpu.sync_copy(dst_hbm.at[wid, pl.ds(grp * GW, GW)], dst_v)
        pltpu.async_copy(y_hbm.at[src_v.at[0]], rows0, sem0)
        pltpu.async_copy(y_hbm.at[src_v.at[1]], rows1, sem1)

        @pl.loop(0, GW - 2, step=2)
        def _(g):
            for b in range(2):
                c = g + b
                pltpu.make_async_copy(y_hbm.at[src_v.at[c]], rows[b],
                                      sems[b]).wait()
                pltpu.sync_copy(rows[b], acc.at[dst_v.at[c]], add=True)
                pltpu.async_copy(y_hbm.at[src_v.at[c + 2]], rows[b], sems[b])

        for b in range(2):
            c = GW - 2 + b
            pltpu.make_async_copy(y_hbm.at[src_v.at[c]], rows[b],
                                  sems[b]).wait()
            pltpu.sync_copy(rows[b], acc.at[dst_v.at[c]], add=True)

    plsc.subcore_barrier()
    pltpu.sync_copy(acc.at[pl.ds(sid * SDEG, SDEG)],
                    out_hbm.at[cid, pl.ds(sid * SDEG, SDEG)])


def _bn_relu_w(aggp_ref, y_ref, degp_ref, cb_ref, g_ref, bb_ref):
    dinv = _deg_inv(degp_ref)
    t = dinv * (aggp_ref[0, :N, :] + aggp_ref[1, :N, :] + y_ref[:N, :]) + cb_ref[...]
    mu = jnp.mean(t, axis=0, keepdims=True)
    dev = t - mu
    var = jnp.mean(dev * dev, axis=0, keepdims=True)
    h = jnp.maximum(dev * lax.rsqrt(var + 1e-5) * g_ref[...] + bb_ref[...], 0.0)
    return h, dinv


def _layer_body_w(aggp_ref, y_ref, degp_ref, cb_ref, g_ref, bb_ref, wn_ref, out_ref):
    h, dinv = _bn_relu_w(aggp_ref, y_ref, degp_ref, cb_ref, g_ref, bb_ref)
    yn = jnp.dot(dinv * h, wn_ref[...], preferred_element_type=jnp.float32)
    out_ref[...] = jnp.concatenate(
        [yn, jnp.zeros((YPAD - N, D), jnp.float32)], axis=0)


def _last_body_w(aggp_ref, y_ref, degp_ref, cb_ref, g_ref, bb_ref,
                 wout_ref, bout_ref, out_ref):
    h, _ = _bn_relu_w(aggp_ref, y_ref, degp_ref, cb_ref, g_ref, bb_ref)
    s = jnp.sum(h, axis=0, keepdims=True) * (1.0 / N)
    out_ref[...] = jnp.dot(s, wout_ref[...],
                           preferred_element_type=jnp.float32) + bout_ref[...]


def _deg_inv(degp_ref):
    deg = degp_ref[0, :N, 0:1] + degp_ref[1, :N, 0:1] + 1.0
    return lax.rsqrt(deg)


def _agg(aggp_ref):
    return jnp.concatenate(
        [aggp_ref[0, :DSTH, :], aggp_ref[1, :N - DSTH, :]], axis=0)


def _emb_body(x_ref, wemb_ref, bemb_ref, w0_ref, degp_ref, y0_ref):
    h = jnp.dot(x_ref[...], wemb_ref[...],
                preferred_element_type=jnp.float32) + bemb_ref[...]
    xw = jnp.dot(h, w0_ref[...], preferred_element_type=jnp.float32)
    y0 = xw * _deg_inv(degp_ref)
    y0_ref[...] = jnp.concatenate(
        [y0, jnp.zeros((YPAD - N, D), jnp.float32)], axis=0)


def _bn_relu(aggp_ref, y_ref, degp_ref, cb_ref, g_ref, bb_ref):
    dinv = _deg_inv(degp_ref)
    t = dinv * (_agg(aggp_ref) + y_ref[:N, :]) + cb_ref[...]
    mu = jnp.mean(t, axis=0, keepdims=True)
    dev = t - mu
    var = jnp.mean(dev * dev, axis=0, keepdims=True)
    h = jnp.maximum(dev * lax.rsqrt(var + 1e-5) * g_ref[...] + bb_ref[...], 0.0)
    return h, dinv


def _layer_body(aggp_ref, y_ref, degp_ref, cb_ref, g_ref, bb_ref, wn_ref, out_ref):
    h, dinv = _bn_relu(aggp_ref, y_ref, degp_ref, cb_ref, g_ref, bb_ref)
    yn = jnp.dot(dinv * h, wn_ref[...], preferred_element_type=jnp.float32)
    out_ref[...] = jnp.concatenate(
        [yn, jnp.zeros((YPAD - N, D), jnp.float32)], axis=0)


def _last_body(aggp_ref, y_ref, degp_ref, cb_ref, g_ref, bb_ref,
               wout_ref, bout_ref, out_ref):
    h, _ = _bn_relu(aggp_ref, y_ref, degp_ref, cb_ref, g_ref, bb_ref)
    s = jnp.sum(h, axis=0, keepdims=True) * (1.0 / N)
    out_ref[...] = jnp.dot(s, wout_ref[...],
                           preferred_element_type=jnp.float32) + bout_ref[...]


def kernel(x, edge_index, W_emb, b_emb, conv_w, conv_b, bn_g, bn_b, W_out, b_out):
    src = edge_index[0]
    dst = edge_index[1]
    E = src.shape[0]

    pad = EP - E
    dst_t = jnp.concatenate([dst, jnp.full((pad,), N, jnp.int32)]).reshape(
        NW, KDEG, B)
    onesD = jnp.ones((B, D), jnp.float32)
    zerosDeg = jnp.zeros((SDEG, D), jnp.float32)
    zerosAcc = jnp.zeros((ASTRIPE, D), jnp.float32)

    third = src // THIRD
    half = (dst >= DSTH).astype(jnp.int32)
    bucket = third * 2 + half
    src_ext = jnp.concatenate([src - third * THIRD, jnp.zeros((1,), jnp.int32)])
    dst_ext = jnp.concatenate([dst - half * DSTH, jnp.full((1,), GARB, jnp.int32)])
    cap = 16 * C * B
    rs, rd = [], []
    for bkt in range(6):
        sel = jnp.nonzero(bucket == bkt, size=cap, fill_value=E)[0]
        rs.append(src_ext[sel])
        rd.append(dst_ext[sel])
    src_r = jnp.stack(rs).reshape(3, 2, 16, C, B)
    dst_r = jnp.stack(rd).reshape(3, 2, 16, C, B)

    src_t = jnp.concatenate([src, jnp.zeros((pad,), jnp.int32)]).reshape(
        NW, KDEG, B)

    degp = _deg_kernel(dst_t, onesD, zerosDeg)

    y0 = pl.pallas_call(
        _emb_body, out_shape=jax.ShapeDtypeStruct((YPAD, D), jnp.float32))(
            x, W_emb, b_emb.reshape(1, D), conv_w[0], degp)

    def _fast(y):
        out = None
        for l in range(4):
            aggp = _scatter_kernel(y, src_r, dst_r, zerosAcc)
            if l < 3:
                y = pl.pallas_call(
                    _layer_body,
                    out_shape=jax.ShapeDtypeStruct((YPAD, D), jnp.float32))(
                        aggp, y, degp, conv_b[l].reshape(1, D),
                        bn_g[l].reshape(1, D), bn_b[l].reshape(1, D),
                        conv_w[l + 1])
            else:
                out = pl.pallas_call(
                    _last_body,
                    out_shape=jax.ShapeDtypeStruct((1, 1), jnp.float32))(
                        aggp, y, degp, conv_b[l].reshape(1, D),
                        bn_g[l].reshape(1, D), bn_b[l].reshape(1, D),
                        W_out, b_out.reshape(1, 1))
        return out

    def _safe(y):
        out = None
        for l in range(4):
            aggp = _scatter_wide(y, src_t, dst_t, zerosDeg)
            if l < 3:
                y = pl.pallas_call(
                    _layer_body_w,
                    out_shape=jax.ShapeDtypeStruct((YPAD, D), jnp.float32))(
                        aggp, y, degp, conv_b[l].reshape(1, D),
                        bn_g[l].reshape(1, D), bn_b[l].reshape(1, D),
                        conv_w[l + 1])
            else:
                out = pl.pallas_call(
                    _last_body_w,
                    out_shape=jax.ShapeDtypeStruct((1, 1), jnp.float32))(
                        aggp, y, degp, conv_b[l].reshape(1, D),
                        bn_g[l].reshape(1, D), bn_b[l].reshape(1, D),
                        W_out, b_out.reshape(1, 1))
        return out

    fits = jnp.all(jnp.stack(
        [jnp.sum(bucket == bkt) for bkt in range(6)]) <= cap)
    return lax.cond(fits, _fast, _safe, y0)


# speedup vs baseline: 3.0694x; 1.2363x over previous
import functools

import jax
import jax.numpy as jnp
from jax import lax
from jax.experimental import pallas as pl
from jax.experimental.pallas import tpu as pltpu
from jax.experimental.pallas import tpu_sc as plsc

N = 10000
D = 128
NW = 32
B = 128

KDEG = 80
NPD = 10112
SDEG = NPD // 16
EP = NW * KDEG * B

SRCH = 5000
HTAB = 5248
YPAD = 10368
TSTRIPE = HTAB // 16
DSTH = 5000
ACCR = 5248
ASTRIPE = ACCR // 16
GARB = 5040
C = 40

_MESH = plsc.VectorSubcoreMesh(core_axis_name="c", subcore_axis_name="s")


@functools.partial(
    pl.kernel,
    out_type=jax.ShapeDtypeStruct((2, NPD, D), jnp.float32),
    mesh=_MESH,
    scratch_types=[
        pltpu.VMEM((KDEG, B), jnp.int32),
        pltpu.VMEM((B, D), jnp.float32),
        pltpu.VMEM_SHARED((NPD, D), jnp.float32),
    ],
)
def _deg_kernel(dst_hbm, ones_hbm, zeros_hbm, out_hbm, dst_v, ones_v, acc):
    cid = lax.axis_index("c")
    sid = lax.axis_index("s")
    wid = cid * 16 + sid
    pltpu.sync_copy(dst_hbm.at[wid], dst_v)
    pltpu.sync_copy(ones_hbm, ones_v)
    pltpu.sync_copy(zeros_hbm, acc.at[pl.ds(sid * SDEG, SDEG)])
    plsc.subcore_barrier()

    @pl.loop(0, KDEG)
    def _(c):
        pltpu.sync_copy(ones_v, acc.at[dst_v.at[c]], add=True)

    plsc.subcore_barrier()
    pltpu.sync_copy(acc.at[pl.ds(sid * SDEG, SDEG)],
                    out_hbm.at[cid, pl.ds(sid * SDEG, SDEG)])


@functools.partial(
    pl.kernel,
    out_type=jax.ShapeDtypeStruct((2, ACCR, D), jnp.float32),
    mesh=_MESH,
    scratch_types=[
        pltpu.VMEM((C, B), jnp.int32),
        pltpu.VMEM((C, B), jnp.int32),
        pltpu.VMEM((B, D), jnp.float32),
        pltpu.VMEM((B, D), jnp.float32),
        pltpu.VMEM_SHARED((HTAB, D), jnp.float32),
        pltpu.VMEM_SHARED((ACCR, D), jnp.float32),
        pltpu.SemaphoreType.DMA,
        pltpu.SemaphoreType.DMA,
    ],
)
def _scatter_kernel(y_hbm, src_hbm, dst_hbm, zeros_hbm, out_hbm,
                    src_v, dst_v, rows0, rows1, ysh, acc, sem0, sem1):
    cid = lax.axis_index("c")
    sid = lax.axis_index("s")
    pltpu.sync_copy(zeros_hbm, acc.at[pl.ds(sid * ASTRIPE, ASTRIPE)])
    rows = (rows0, rows1)
    sems = (sem0, sem1)

    for p in range(2):
        pltpu.sync_copy(y_hbm.at[pl.ds(p * SRCH + sid * TSTRIPE, TSTRIPE)],
                        ysh.at[pl.ds(sid * TSTRIPE, TSTRIPE)])
        pltpu.sync_copy(src_hbm.at[p, cid, sid], src_v)
        pltpu.sync_copy(dst_hbm.at[p, cid, sid], dst_v)
        plsc.subcore_barrier()

        pltpu.async_copy(ysh.at[src_v.at[0]], rows0, sem0)
        pltpu.async_copy(ysh.at[src_v.at[1]], rows1, sem1)

        @pl.loop(0, C - 2, step=2)
        def _(g):
            for b in range(2):
                c = g + b
                pltpu.make_async_copy(ysh.at[src_v.at[c]], rows[b],
                                      sems[b]).wait()
                pltpu.sync_copy(rows[b], acc.at[dst_v.at[c]], add=True)
                pltpu.async_copy(ysh.at[src_v.at[c + 2]], rows[b], sems[b])

        for b in range(2):
            c = C - 2 + b
            pltpu.make_async_copy(ysh.at[src_v.at[c]], rows[b],
                                  sems[b]).wait()
            pltpu.sync_copy(rows[b], acc.at[dst_v.at[c]], add=True)

        plsc.subcore_barrier()

    pltpu.sync_copy(acc.at[pl.ds(sid * ASTRIPE, ASTRIPE)],
                    out_hbm.at[cid, pl.ds(sid * ASTRIPE, ASTRIPE)])


GW = 16


@functools.partial(
    pl.kernel,
    out_type=jax.ShapeDtypeStruct((2, NPD, D), jnp.float32),
    mesh=_MESH,
    scratch_types=[
        pltpu.VMEM((GW, B), jnp.int32),
        pltpu.VMEM((GW, B), jnp.int32),
        pltpu.VMEM((B, D), jnp.float32),
        pltpu.VMEM((B, D), jnp.float32),
        pltpu.VMEM_SHARED((NPD, D), jnp.float32),
        pltpu.SemaphoreType.DMA,
        pltpu.SemaphoreType.DMA,
    ],
)
def _scatter_wide(y_hbm, src_hbm, dst_hbm, zeros_hbm, out_hbm,
                  src_v, dst_v, rows0, rows1, acc, sem0, sem1):
    cid = lax.axis_index("c")
    sid = lax.axis_index("s")
    wid = cid * 16 + sid
    pltpu.sync_copy(zeros_hbm, acc.at[pl.ds(sid * SDEG, SDEG)])
    plsc.subcore_barrier()

    rows = (rows0, rows1)
    sems = (sem0, sem1)
    for grp in range(KDEG // GW):
        pltpu.sync_copy(src_hbm.at[wid, pl.ds(grp * GW, GW)], src_v)
        pltpu.sync_copy(dst_hbm.at[wid, pl.ds(grp * GW, GW)], dst_v)
        pltpu.async_copy(y_hbm.at[src_v.at[0]], rows0, sem0)
        pltpu.async_copy(y_hbm.at[src_v.at[1]], rows1, sem1)

        @pl.loop(0, GW - 2, step=2)
        def _(g):
            for b in range(2):
                c = g + b
                pltpu.make_async_copy(y_hbm.at[src_v.at[c]], rows[b],
                                      sems[b]).wait()
                pltpu.sync_copy(rows[b], acc.at[dst_v.at[c]], add=True)
                pltpu.async_copy(y_hbm.at[src_v.at[c + 2]], rows[b], sems[b])

        for b in range(2):
            c = GW - 2 + b
            pltpu.make_async_copy(y_hbm.at[src_v.at[c]], rows[b],
                                  sems[b]).wait()
            pltpu.sync_copy(rows[b], acc.at[dst_v.at[c]], add=True)

    plsc.subcore_barrier()
    pltpu.sync_copy(acc.at[pl.ds(sid * SDEG, SDEG)],
                    out_hbm.at[cid, pl.ds(sid * SDEG, SDEG)])


def _bn_relu_w(aggp_ref, y_ref, degp_ref, cb_ref, g_ref, bb_ref):
    dinv = _deg_inv(degp_ref)
    t = dinv * (aggp_ref[0, :N, :] + aggp_ref[1, :N, :] + y_ref[:N, :]) + cb_ref[...]
    mu = jnp.mean(t, axis=0, keepdims=True)
    dev = t - mu
    var = jnp.mean(dev * dev, axis=0, keepdims=True)
    h = jnp.maximum(dev * lax.rsqrt(var + 1e-5) * g_ref[...] + bb_ref[...], 0.0)
    return h, dinv


def _layer_body_w(aggp_ref, y_ref, degp_ref, cb_ref, g_ref, bb_ref, wn_ref, out_ref):
    h, dinv = _bn_relu_w(aggp_ref, y_ref, degp_ref, cb_ref, g_ref, bb_ref)
    yn = jnp.dot(dinv * h, wn_ref[...], preferred_element_type=jnp.float32)
    out_ref[...] = jnp.concatenate(
        [yn, jnp.zeros((YPAD - N, D), jnp.float32)], axis=0)


def _last_body_w(aggp_ref, y_ref, degp_ref, cb_ref, g_ref, bb_ref,
                 wout_ref, bout_ref, out_ref):
    h, _ = _bn_relu_w(aggp_ref, y_ref, degp_ref, cb_ref, g_ref, bb_ref)
    s = jnp.sum(h, axis=0, keepdims=True) * (1.0 / N)
    out_ref[...] = jnp.dot(s, wout_ref[...],
                           preferred_element_type=jnp.float32) + bout_ref[...]


def _deg_inv(degp_ref):
    deg = degp_ref[0, :N, 0:1] + degp_ref[1, :N, 0:1] + 1.0
    return lax.rsqrt(deg)


def _agg(aggp_ref):
    return jnp.concatenate(
        [aggp_ref[0, :DSTH, :], aggp_ref[1, :N - DSTH, :]], axis=0)


def _emb_body(x_ref, wemb_ref, bemb_ref, w0_ref, degp_ref, y0_ref):
    h = jnp.dot(x_ref[...], wemb_ref[...],
                preferred_element_type=jnp.float32) + bemb_ref[...]
    xw = jnp.dot(h, w0_ref[...], preferred_element_type=jnp.float32)
    y0 = xw * _deg_inv(degp_ref)
    y0_ref[...] = jnp.concatenate(
        [y0, jnp.zeros((YPAD - N, D), jnp.float32)], axis=0)


def _bn_relu(aggp_ref, y_ref, degp_ref, cb_ref, g_ref, bb_ref):
    dinv = _deg_inv(degp_ref)
    t = dinv * (_agg(aggp_ref) + y_ref[:N, :]) + cb_ref[...]
    mu = jnp.mean(t, axis=0, keepdims=True)
    dev = t - mu
    var = jnp.mean(dev * dev, axis=0, keepdims=True)
    h = jnp.maximum(dev * lax.rsqrt(var + 1e-5) * g_ref[...] + bb_ref[...], 0.0)
    return h, dinv


def _layer_body(aggp_ref, y_ref, degp_ref, cb_ref, g_ref, bb_ref, wn_ref, out_ref):
    h, dinv = _bn_relu(aggp_ref, y_ref, degp_ref, cb_ref, g_ref, bb_ref)
    yn = jnp.dot(dinv * h, wn_ref[...], preferred_element_type=jnp.float32)
    out_ref[...] = jnp.concatenate(
        [yn, jnp.zeros((YPAD - N, D), jnp.float32)], axis=0)


def _last_body(aggp_ref, y_ref, degp_ref, cb_ref, g_ref, bb_ref,
               wout_ref, bout_ref, out_ref):
    h, _ = _bn_relu(aggp_ref, y_ref, degp_ref, cb_ref, g_ref, bb_ref)
    s = jnp.sum(h, axis=0, keepdims=True) * (1.0 / N)
    out_ref[...] = jnp.dot(s, wout_ref[...],
                           preferred_element_type=jnp.float32) + bout_ref[...]


def kernel(x, edge_index, W_emb, b_emb, conv_w, conv_b, bn_g, bn_b, W_out, b_out):
    src = edge_index[0]
    dst = edge_index[1]
    E = src.shape[0]

    pad = EP - E
    dst_t = jnp.concatenate([dst, jnp.full((pad,), N, jnp.int32)]).reshape(
        NW, KDEG, B)
    onesD = jnp.ones((B, D), jnp.float32)
    zerosDeg = jnp.zeros((SDEG, D), jnp.float32)
    zerosAcc = jnp.zeros((ASTRIPE, D), jnp.float32)

    shalf = (src >= SRCH).astype(jnp.int32)
    half = (dst >= DSTH).astype(jnp.int32)
    bucket = shalf * 2 + half
    src_ext = jnp.concatenate([src - shalf * SRCH, jnp.zeros((1,), jnp.int32)])
    dst_ext = jnp.concatenate([dst - half * DSTH, jnp.full((1,), GARB, jnp.int32)])
    cap = 16 * C * B
    rs, rd = [], []
    for bkt in range(4):
        sel = jnp.nonzero(bucket == bkt, size=cap, fill_value=E)[0]
        rs.append(src_ext[sel])
        rd.append(dst_ext[sel])
    src_r = jnp.stack(rs).reshape(2, 2, 16, C, B)
    dst_r = jnp.stack(rd).reshape(2, 2, 16, C, B)

    src_t = jnp.concatenate([src, jnp.zeros((pad,), jnp.int32)]).reshape(
        NW, KDEG, B)

    degp = _deg_kernel(dst_t, onesD, zerosDeg)

    y0 = pl.pallas_call(
        _emb_body, out_shape=jax.ShapeDtypeStruct((YPAD, D), jnp.float32))(
            x, W_emb, b_emb.reshape(1, D), conv_w[0], degp)

    def _fast(y):
        out = None
        for l in range(4):
            aggp = _scatter_kernel(y, src_r, dst_r, zerosAcc)
            if l < 3:
                y = pl.pallas_call(
                    _layer_body,
                    out_shape=jax.ShapeDtypeStruct((YPAD, D), jnp.float32))(
                        aggp, y, degp, conv_b[l].reshape(1, D),
                        bn_g[l].reshape(1, D), bn_b[l].reshape(1, D),
                        conv_w[l + 1])
            else:
                out = pl.pallas_call(
                    _last_body,
                    out_shape=jax.ShapeDtypeStruct((1, 1), jnp.float32))(
                        aggp, y, degp, conv_b[l].reshape(1, D),
                        bn_g[l].reshape(1, D), bn_b[l].reshape(1, D),
                        W_out, b_out.reshape(1, 1))
        return out

    def _safe(y):
        out = None
        for l in range(4):
            aggp = _scatter_wide(y, src_t, dst_t, zerosDeg)
            if l < 3:
                y = pl.pallas_call(
                    _layer_body_w,
                    out_shape=jax.ShapeDtypeStruct((YPAD, D), jnp.float32))(
                        aggp, y, degp, conv_b[l].reshape(1, D),
                        bn_g[l].reshape(1, D), bn_b[l].reshape(1, D),
                        conv_w[l + 1])
            else:
                out = pl.pallas_call(
                    _last_body_w,
                    out_shape=jax.ShapeDtypeStruct((1, 1), jnp.float32))(
                        aggp, y, degp, conv_b[l].reshape(1, D),
                        bn_g[l].reshape(1, D), bn_b[l].reshape(1, D),
                        W_out, b_out.reshape(1, 1))
        return out

    fits = jnp.all(jnp.stack(
        [jnp.sum(bucket == bkt) for bkt in range(4)]) <= cap)
    return lax.cond(fits, _fast, _safe, y0)
